# baseline (speedup 1.0000x reference)
import functools

import jax
import jax.numpy as jnp
from jax import lax
from jax.experimental import pallas as pl
from jax.experimental.pallas import tpu as pltpu
from jax.experimental.pallas import tpu_sc as plsc

EMB = 8
N_SP = 20
N_DN = 10
D_OUT = N_SP * EMB + N_DN * EMB
BATCH = 16384
NC = 2
NS = 16
NW = NC * NS
RPW = BATCH // NW
CHUNK = 128
GROUPS = CHUNK // 16


def _body(offs, sp_ref, dn_ref, tb_ref, wb_ref, out_ref,
          sp_v, dn_v, tb_v, wb_v, ob_v):
    cid = lax.axis_index("c")
    sid = lax.axis_index("s")
    wid = sid * NC + cid
    base = wid * RPW
    pltpu.sync_copy(sp_ref.at[pl.ds(base * N_SP, RPW * N_SP)], sp_v)
    pltpu.sync_copy(dn_ref.at[pl.ds(base * N_DN, RPW * N_DN)], dn_v)
    pltpu.sync_copy(tb_ref, tb_v)
    pltpu.sync_copy(wb_ref, wb_v)
    iota = lax.iota(jnp.int32, 16)
    iota_sp = iota * N_SP
    iota_dn = iota * N_DN
    iota_out = iota * D_OUT

    def chunk_body(chunk, carry):
        r0c = chunk * CHUNK

        def group(g, carry):
            r0 = r0c + g * 16
            rows_sp = iota_sp + r0 * N_SP
            rows_dn = iota_dn + r0 * N_DN
            rows_out = iota_out + g * (16 * D_OUT)
            for i in range(N_SP):
                sidx = plsc.load_gather(sp_v, [rows_sp + i])
                t8 = sidx * EMB
                for j in range(EMB):
                    v = plsc.load_gather(tb_v, [t8 + (offs[i] * EMB + j)])
                    plsc.store_scatter(ob_v, [rows_out + (EMB * i + j)], v)
            dcols = [plsc.load_gather(dn_v, [rows_dn + k])
                     for k in range(N_DN)]
            for o in range(N_DN * EMB):
                base_w = o * (N_DN + 1) * 16
                acc = wb_v[pl.ds(base_w, 16)]
                for k in range(N_DN):
                    acc = acc + dcols[k] * wb_v[pl.ds(base_w + (k + 1) * 16, 16)]
                plsc.store_scatter(ob_v, [rows_out + (N_SP * EMB + o)], acc)
            return carry

        lax.fori_loop(0, GROUPS, group, 0)
        pltpu.sync_copy(
            ob_v, out_ref.at[pl.ds((base + r0c) * D_OUT, CHUNK * D_OUT)])
        return carry

    lax.fori_loop(0, RPW // CHUNK, chunk_body, 0)


def kernel(sparse, dense, W, b, emb0, emb1, emb2, emb3, emb4, emb5, emb6,
           emb7, emb8, emb9, emb10, emb11, emb12, emb13, emb14, emb15,
           emb16, emb17, emb18, emb19):
    embs = [emb0, emb1, emb2, emb3, emb4, emb5, emb6, emb7, emb8, emb9,
            emb10, emb11, emb12, emb13, emb14, emb15, emb16, emb17, emb18,
            emb19]
    offs = []
    t = 0
    for e in embs:
        offs.append(t)
        t += e.shape[0]
    table = jnp.concatenate(embs, axis=0).reshape(-1)
    wb = jnp.broadcast_to(
        jnp.concatenate([b[:, None], W], axis=1)[:, :, None],
        (N_DN * EMB, N_DN + 1, 16)).reshape(-1)

    mesh = plsc.VectorSubcoreMesh(core_axis_name="c", subcore_axis_name="s")
    k = pl.kernel(
        functools.partial(_body, tuple(offs)),
        mesh=mesh,
        compiler_params=pltpu.CompilerParams(needs_layout_passes=False),
        out_type=jax.ShapeDtypeStruct((BATCH * D_OUT,), jnp.float32),
        scratch_types=[
            pltpu.VMEM((RPW * N_SP,), jnp.int32),
            pltpu.VMEM((RPW * N_DN,), jnp.float32),
            pltpu.VMEM((t * EMB,), jnp.float32),
            pltpu.VMEM((N_DN * EMB * (N_DN + 1) * 16,), jnp.float32),
            pltpu.VMEM((CHUNK * D_OUT,), jnp.float32),
        ],
    )
    out = k(sparse.reshape(-1), dense.reshape(-1), table, wb)
    return out.reshape(BATCH, D_OUT)


# speedup vs baseline: 1.7902x; 1.7902x over previous
import functools

import jax
import jax.numpy as jnp
from jax import lax
from jax.experimental import pallas as pl
from jax.experimental.pallas import tpu as pltpu
from jax.experimental.pallas import tpu_sc as plsc

EMB = 8
N_SP = 20
N_DN = 10
D_SP = N_SP * EMB
D_DN = N_DN * EMB
D_OUT = D_SP + D_DN
BATCH = 16384
NC = 2
NS = 16
NW = NC * NS
RPW = BATCH // NW
CHUNK = 128
GROUPS = CHUNK // 16
TB = 2048


def _dense_body(dense_ref, wt_ref, b_ref, out_ref):
    out_ref[...] = (
        jnp.dot(dense_ref[...], wt_ref[...],
                preferred_element_type=jnp.float32)
        + b_ref[...]
    )


def _sc_body(offs, sp_ref, tb_ref, dp_ref, out_ref, sp_v, tb_v, ob_v):
    cid = lax.axis_index("c")
    sid = lax.axis_index("s")
    wid = sid * NC + cid
    base = wid * RPW
    pltpu.sync_copy(sp_ref.at[pl.ds(base * N_SP, RPW * N_SP)], sp_v)
    pltpu.sync_copy(tb_ref, tb_v)
    iota = lax.iota(jnp.int32, 16)
    iota_sp = iota * N_SP

    def chunk_body(chunk, carry):
        r0c = chunk * CHUNK

        def group(g, carry):
            r0 = r0c + g * 16
            rows_sp = iota_sp + r0 * N_SP
            rows_ob = iota + g * 16
            for i in range(N_SP):
                sidx = plsc.load_gather(sp_v, [rows_sp + i])
                t8 = sidx * EMB
                for j in range(EMB):
                    v = plsc.load_gather(tb_v, [t8 + (offs[i] * EMB + j)])
                    ci = jnp.full((16,), EMB * i + j, jnp.int32)
                    plsc.store_scatter(ob_v, [rows_ob, ci], v)
            return carry

        rows = pl.ds(base + r0c, CHUNK)
        pltpu.sync_copy(dp_ref.at[rows, :], ob_v.at[:, pl.ds(D_SP, D_DN)])
        lax.fori_loop(0, GROUPS, group, 0)
        pltpu.sync_copy(ob_v, out_ref.at[rows, :])
        return carry

    lax.fori_loop(0, RPW // CHUNK, chunk_body, 0)


def kernel(sparse, dense, W, b, emb0, emb1, emb2, emb3, emb4, emb5, emb6,
           emb7, emb8, emb9, emb10, emb11, emb12, emb13, emb14, emb15,
           emb16, emb17, emb18, emb19):
    embs = [emb0, emb1, emb2, emb3, emb4, emb5, emb6, emb7, emb8, emb9,
            emb10, emb11, emb12, emb13, emb14, emb15, emb16, emb17, emb18,
            emb19]
    offs = []
    t = 0
    for e in embs:
        offs.append(t)
        t += e.shape[0]
    table = jnp.concatenate(embs, axis=0).reshape(-1)

    dp = pl.pallas_call(
        _dense_body,
        grid=(BATCH // TB,),
        in_specs=[
            pl.BlockSpec((TB, N_DN), lambda i: (i, 0)),
            pl.BlockSpec((N_DN, D_DN), lambda i: (0, 0)),
            pl.BlockSpec((1, D_DN), lambda i: (0, 0)),
        ],
        out_specs=pl.BlockSpec((TB, D_DN), lambda i: (i, 0)),
        out_shape=jax.ShapeDtypeStruct((BATCH, D_DN), jnp.float32),
    )(dense, W.T, b[None, :])

    mesh = plsc.VectorSubcoreMesh(core_axis_name="c", subcore_axis_name="s")
    k = pl.kernel(
        functools.partial(_sc_body, tuple(offs)),
        mesh=mesh,
        compiler_params=pltpu.CompilerParams(
            needs_layout_passes=False, use_tc_tiling_on_sc=False),
        out_type=jax.ShapeDtypeStruct((BATCH, D_OUT), jnp.float32),
        scratch_types=[
            pltpu.VMEM((RPW * N_SP,), jnp.int32),
            pltpu.VMEM((t * EMB,), jnp.float32),
            pltpu.VMEM((CHUNK, D_OUT), jnp.float32),
        ],
    )
    return k(sparse.reshape(-1), table, dp)


# speedup vs baseline: 2.1371x; 1.1938x over previous
import functools

import jax
import jax.numpy as jnp
from jax import lax
from jax.experimental import pallas as pl
from jax.experimental.pallas import tpu as pltpu
from jax.experimental.pallas import tpu_sc as plsc

EMB = 8
N_SP = 20
N_DN = 10
D_SP = N_SP * EMB
D_DN = N_DN * EMB
D_OUT = D_SP + D_DN
BATCH = 16384
NC = 2
NS = 16
NW = NC * NS
RPW = BATCH // NW
CHUNK = 128
GROUPS = CHUNK // 16
TB = 2048


def _dense_body(dense_ref, wt_ref, b_ref, out_ref):
    out_ref[...] = (
        jnp.dot(dense_ref[...], wt_ref[...],
                preferred_element_type=jnp.float32)
        + b_ref[...]
    )


def _sc_body(offs, sp_ref, tb_ref, dp_ref, out_ref, sp_v, tb_v, ob0, ob1,
             sem0, sem1):
    cid = lax.axis_index("c")
    sid = lax.axis_index("s")
    wid = sid * NC + cid
    base = wid * RPW
    pltpu.sync_copy(sp_ref.at[pl.ds(base * N_SP, RPW * N_SP)], sp_v)
    pltpu.sync_copy(tb_ref, tb_v)
    iota = lax.iota(jnp.int32, 16)
    iota_sp = iota * N_SP

    def do_chunk(chunk, ob_v, sem, first):
        r0c = chunk * CHUNK
        rows = pl.ds(base + r0c, CHUNK)
        if not first:
            pltpu.make_async_copy(ob_v, out_ref.at[rows, :], sem).wait()
        pltpu.sync_copy(dp_ref.at[rows, :], ob_v.at[:, pl.ds(D_SP, D_DN)])

        def group(g, carry):
            r0 = r0c + g * 16
            rows_sp = iota_sp + r0 * N_SP
            rows_ob = iota + g * 16
            sidx8 = [plsc.load_gather(sp_v, [rows_sp + i]) * EMB
                     for i in range(N_SP)]
            for i in range(N_SP):
                t8 = sidx8[i]
                vs = [plsc.load_gather(tb_v, [t8 + (offs[i] * EMB + j)])
                      for j in range(EMB)]
                for j in range(EMB):
                    ci = jnp.full((16,), EMB * i + j, jnp.int32)
                    plsc.store_scatter(ob_v, [rows_ob, ci], vs[j])
            return carry

        lax.fori_loop(0, GROUPS, group, 0)
        pltpu.async_copy(ob_v, out_ref.at[rows, :], sem)

    n_chunks = RPW // CHUNK
    for c in range(n_chunks):
        do_chunk(c, (ob0, ob1)[c % 2], (sem0, sem1)[c % 2], c < 2)
    for c in range(n_chunks - 2, n_chunks):
        pltpu.make_async_copy(
            (ob0, ob1)[c % 2],
            out_ref.at[pl.ds(base + c * CHUNK, CHUNK), :],
            (sem0, sem1)[c % 2]).wait()


def kernel(sparse, dense, W, b, emb0, emb1, emb2, emb3, emb4, emb5, emb6,
           emb7, emb8, emb9, emb10, emb11, emb12, emb13, emb14, emb15,
           emb16, emb17, emb18, emb19):
    embs = [emb0, emb1, emb2, emb3, emb4, emb5, emb6, emb7, emb8, emb9,
            emb10, emb11, emb12, emb13, emb14, emb15, emb16, emb17, emb18,
            emb19]
    offs = []
    t = 0
    for e in embs:
        offs.append(t)
        t += e.shape[0]
    table = jnp.concatenate(embs, axis=0).reshape(-1)

    dp = pl.pallas_call(
        _dense_body,
        grid=(BATCH // TB,),
        in_specs=[
            pl.BlockSpec((TB, N_DN), lambda i: (i, 0)),
            pl.BlockSpec((N_DN, D_DN), lambda i: (0, 0)),
            pl.BlockSpec((1, D_DN), lambda i: (0, 0)),
        ],
        out_specs=pl.BlockSpec((TB, D_DN), lambda i: (i, 0)),
        out_shape=jax.ShapeDtypeStruct((BATCH, D_DN), jnp.float32),
    )(dense, W.T, b[None, :])

    mesh = plsc.VectorSubcoreMesh(core_axis_name="c", subcore_axis_name="s")
    k = pl.kernel(
        functools.partial(_sc_body, tuple(offs)),
        mesh=mesh,
        compiler_params=pltpu.CompilerParams(
            needs_layout_passes=False, use_tc_tiling_on_sc=False),
        out_type=jax.ShapeDtypeStruct((BATCH, D_OUT), jnp.float32),
        scratch_types=[
            pltpu.VMEM((RPW * N_SP,), jnp.int32),
            pltpu.VMEM((t * EMB,), jnp.float32),
            pltpu.VMEM((CHUNK, D_OUT), jnp.float32),
            pltpu.VMEM((CHUNK, D_OUT), jnp.float32),
            pltpu.SemaphoreType.DMA,
            pltpu.SemaphoreType.DMA,
        ],
    )
    return k(sparse.reshape(-1), table, dp)


# speedup vs baseline: 2.1701x; 1.0154x over previous
import functools

import jax
import jax.numpy as jnp
from jax import lax
from jax.experimental import pallas as pl
from jax.experimental.pallas import tpu as pltpu
from jax.experimental.pallas import tpu_sc as plsc

EMB = 8
N_SP = 20
N_DN = 10
D_SP = N_SP * EMB
D_DN = N_DN * EMB
D_OUT = D_SP + D_DN
BATCH = 16384
NC = 2
NS = 16
NW = NC * NS
RPW = BATCH // NW
CHUNK = 128
GROUPS = CHUNK // 16
TB = 2048


def _dense_body(dense_ref, wt_ref, b_ref, out_ref):
    out_ref[...] = (
        jnp.dot(dense_ref[...], wt_ref[...],
                preferred_element_type=jnp.float32)
        + b_ref[...]
    )


D_CT1 = D_OUT - 128


def _sc_body(offs, sp_ref, tb_ref, dp_ref, out_ref, sp_v, tb_v,
             ob0a, ob0b, ob1a, ob1b, sem0a, sem0b, sem1a, sem1b):
    cid = lax.axis_index("c")
    sid = lax.axis_index("s")
    wid = sid * NC + cid
    base = wid * RPW
    pltpu.sync_copy(sp_ref.at[pl.ds(base * N_SP, RPW * N_SP)], sp_v)
    pltpu.sync_copy(tb_ref, tb_v)
    iota = lax.iota(jnp.int32, 16)
    iota_sp = iota * N_SP

    def do_chunk(chunk, ct0, ct1, s0, s1, first):
        r0c = chunk * CHUNK
        rows = pl.ds(base + r0c, CHUNK)
        if not first:
            pltpu.make_async_copy(
                ct0, out_ref.at[rows, pl.ds(0, 128)], s0).wait()
            pltpu.make_async_copy(
                ct1, out_ref.at[rows, pl.ds(128, 128)], s1).wait()
        pltpu.sync_copy(dp_ref.at[rows, :], ct1)

        def group(g, carry):
            r0 = r0c + g * 16
            rows_sp = iota_sp + r0 * N_SP
            rows_ob = iota + g * 16
            sidx8 = [plsc.load_gather(sp_v, [rows_sp + i]) * EMB
                     for i in range(N_SP)]
            for i in range(N_SP):
                t8 = sidx8[i]
                vs = [plsc.load_gather(tb_v, [t8 + (offs[i] * EMB + j)])
                      for j in range(EMB)]
                for j in range(EMB):
                    c = EMB * i + j
                    tgt = ct0 if c < 128 else ct1
                    ci = jnp.full((16,), c % 128, jnp.int32)
                    plsc.store_scatter(tgt, [rows_ob, ci], vs[j])
            return carry

        lax.fori_loop(0, GROUPS, group, 0)
        pltpu.async_copy(ct0, out_ref.at[rows, pl.ds(0, 128)], s0)
        pltpu.async_copy(ct1, out_ref.at[rows, pl.ds(128, 128)], s1)

    bufs = [(ob0a, ob1a, sem0a, sem1a), (ob0b, ob1b, sem0b, sem1b)]
    n_chunks = RPW // CHUNK
    for c in range(n_chunks):
        ct0, ct1, s0, s1 = bufs[c % 2]
        do_chunk(c, ct0, ct1, s0, s1, c < 2)
    for c in range(n_chunks - 2, n_chunks):
        ct0, ct1, s0, s1 = bufs[c % 2]
        rows = pl.ds(base + c * CHUNK, CHUNK)
        pltpu.make_async_copy(ct0, out_ref.at[rows, pl.ds(0, 128)], s0).wait()
        pltpu.make_async_copy(
            ct1, out_ref.at[rows, pl.ds(128, 128)], s1).wait()


def kernel(sparse, dense, W, b, emb0, emb1, emb2, emb3, emb4, emb5, emb6,
           emb7, emb8, emb9, emb10, emb11, emb12, emb13, emb14, emb15,
           emb16, emb17, emb18, emb19):
    embs = [emb0, emb1, emb2, emb3, emb4, emb5, emb6, emb7, emb8, emb9,
            emb10, emb11, emb12, emb13, emb14, emb15, emb16, emb17, emb18,
            emb19]
    offs = []
    t = 0
    for e in embs:
        offs.append(t)
        t += e.shape[0]
    table = jnp.concatenate(embs, axis=0).reshape(-1)
    pad = (-table.shape[0]) % 128
    table = jnp.concatenate([table, jnp.zeros((pad,), jnp.float32)])

    lpad = D_SP - 128
    rpad = 128 - lpad - D_DN
    w_pad = jnp.concatenate([
        jnp.zeros((lpad, N_DN), jnp.float32), W,
        jnp.zeros((rpad, N_DN), jnp.float32)])
    b_pad = jnp.concatenate([
        jnp.zeros((lpad,), jnp.float32), b,
        jnp.zeros((rpad,), jnp.float32)])

    dp = pl.pallas_call(
        _dense_body,
        grid=(BATCH // TB,),
        in_specs=[
            pl.BlockSpec((TB, N_DN), lambda i: (i, 0)),
            pl.BlockSpec((N_DN, 128), lambda i: (0, 0)),
            pl.BlockSpec((1, 128), lambda i: (0, 0)),
        ],
        out_specs=pl.BlockSpec((TB, 128), lambda i: (i, 0)),
        out_shape=jax.ShapeDtypeStruct((BATCH, 128), jnp.float32),
    )(dense, w_pad.T, b_pad[None, :])

    mesh = plsc.VectorSubcoreMesh(core_axis_name="c", subcore_axis_name="s")
    k = pl.kernel(
        functools.partial(_sc_body, tuple(offs)),
        mesh=mesh,
        compiler_params=pltpu.CompilerParams(
            needs_layout_passes=False, use_tc_tiling_on_sc=True),
        out_type=jax.ShapeDtypeStruct((BATCH, 256), jnp.float32),
        scratch_types=[
            pltpu.VMEM((RPW * N_SP,), jnp.int32),
            pltpu.VMEM((table.shape[0],), jnp.float32),
            pltpu.VMEM((CHUNK, 128), jnp.float32),
            pltpu.VMEM((CHUNK, 128), jnp.float32),
            pltpu.VMEM((CHUNK, 128), jnp.float32),
            pltpu.VMEM((CHUNK, 128), jnp.float32),
            pltpu.SemaphoreType.DMA,
            pltpu.SemaphoreType.DMA,
            pltpu.SemaphoreType.DMA,
            pltpu.SemaphoreType.DMA,
        ],
    )
    return k(sparse.reshape(-1), table, dp)[:, :D_OUT]


# speedup vs baseline: 2.2417x; 1.0330x over previous
import functools

import jax
import jax.numpy as jnp
from jax import lax
from jax.experimental import pallas as pl
from jax.experimental.pallas import tpu as pltpu
from jax.experimental.pallas import tpu_sc as plsc

EMB = 8
N_SP = 20
N_DN = 10
D_SP = N_SP * EMB
D_DN = N_DN * EMB
D_OUT = D_SP + D_DN
BATCH = 16384
NC = 2
NS = 16
NW = NC * NS
RPW = BATCH // NW
CHUNK = 128
GROUPS = CHUNK // 16
TB = 8192


def _dense_body(dense_ref, wt_ref, b_ref, out_ref):
    out_ref[...] = (
        jnp.dot(dense_ref[...], wt_ref[...],
                preferred_element_type=jnp.float32)
        + b_ref[...]
    )


D_CT1 = D_OUT - 128


def _sc_body(offs, sp_ref, tb_ref, dp_ref, out_ref, sp_v, tb_v,
             ob0a, ob0b, ob1a, ob1b, sem0a, sem0b, sem1a, sem1b):
    cid = lax.axis_index("c")
    sid = lax.axis_index("s")
    wid = sid * NC + cid
    base = wid * RPW
    pltpu.sync_copy(sp_ref.at[pl.ds(base * N_SP, RPW * N_SP)], sp_v)
    pltpu.sync_copy(tb_ref, tb_v)
    iota = lax.iota(jnp.int32, 16)
    iota_sp = iota * N_SP

    def do_chunk(chunk, ct0, ct1, s0, s1, first):
        r0c = chunk * CHUNK
        rows = pl.ds(base + r0c, CHUNK)
        if not first:
            pltpu.make_async_copy(
                ct0, out_ref.at[rows, pl.ds(0, 128)], s0).wait()
            pltpu.make_async_copy(
                ct1, out_ref.at[rows, pl.ds(128, 128)], s1).wait()
        pltpu.sync_copy(dp_ref.at[rows, :], ct1)

        def group(g, carry):
            r0 = r0c + g * 16
            rows_sp = iota_sp + r0 * N_SP
            rows_ob = iota + g * 16
            sidx8 = [plsc.load_gather(sp_v, [rows_sp + i]) * EMB
                     for i in range(N_SP)]
            for i in range(N_SP):
                t8 = sidx8[i]
                vs = [plsc.load_gather(tb_v, [t8 + (offs[i] * EMB + j)])
                      for j in range(EMB)]
                for j in range(EMB):
                    c = EMB * i + j
                    tgt = ct0 if c < 128 else ct1
                    ci = jnp.full((16,), c % 128, jnp.int32)
                    plsc.store_scatter(tgt, [rows_ob, ci], vs[j])
            return carry

        lax.fori_loop(0, GROUPS, group, 0)
        pltpu.async_copy(ct0, out_ref.at[rows, pl.ds(0, 128)], s0)
        pltpu.async_copy(ct1, out_ref.at[rows, pl.ds(128, 128)], s1)

    bufs = [(ob0a, ob1a, sem0a, sem1a), (ob0b, ob1b, sem0b, sem1b)]
    n_chunks = RPW // CHUNK
    for c in range(n_chunks):
        ct0, ct1, s0, s1 = bufs[c % 2]
        do_chunk(c, ct0, ct1, s0, s1, c < 2)
    for c in range(n_chunks - 2, n_chunks):
        ct0, ct1, s0, s1 = bufs[c % 2]
        rows = pl.ds(base + c * CHUNK, CHUNK)
        pltpu.make_async_copy(ct0, out_ref.at[rows, pl.ds(0, 128)], s0).wait()
        pltpu.make_async_copy(
            ct1, out_ref.at[rows, pl.ds(128, 128)], s1).wait()


def kernel(sparse, dense, W, b, emb0, emb1, emb2, emb3, emb4, emb5, emb6,
           emb7, emb8, emb9, emb10, emb11, emb12, emb13, emb14, emb15,
           emb16, emb17, emb18, emb19):
    embs = [emb0, emb1, emb2, emb3, emb4, emb5, emb6, emb7, emb8, emb9,
            emb10, emb11, emb12, emb13, emb14, emb15, emb16, emb17, emb18,
            emb19]
    offs = []
    t = 0
    for e in embs:
        offs.append(t)
        t += e.shape[0]
    table = jnp.concatenate(embs, axis=0).reshape(-1)
    pad = (-table.shape[0]) % 128
    table = jnp.concatenate([table, jnp.zeros((pad,), jnp.float32)])

    lpad = D_SP - 128
    rpad = 128 - lpad - D_DN
    w_pad = jnp.concatenate([
        jnp.zeros((lpad, N_DN), jnp.float32), W,
        jnp.zeros((rpad, N_DN), jnp.float32)])
    b_pad = jnp.concatenate([
        jnp.zeros((lpad,), jnp.float32), b,
        jnp.zeros((rpad,), jnp.float32)])

    dp = pl.pallas_call(
        _dense_body,
        grid=(BATCH // TB,),
        in_specs=[
            pl.BlockSpec((TB, N_DN), lambda i: (i, 0)),
            pl.BlockSpec((N_DN, 128), lambda i: (0, 0)),
            pl.BlockSpec((1, 128), lambda i: (0, 0)),
        ],
        out_specs=pl.BlockSpec((TB, 128), lambda i: (i, 0)),
        out_shape=jax.ShapeDtypeStruct((BATCH, 128), jnp.float32),
    )(dense, w_pad.T, b_pad[None, :])

    mesh = plsc.VectorSubcoreMesh(core_axis_name="c", subcore_axis_name="s")
    k = pl.kernel(
        functools.partial(_sc_body, tuple(offs)),
        mesh=mesh,
        compiler_params=pltpu.CompilerParams(
            needs_layout_passes=False, use_tc_tiling_on_sc=True),
        out_type=jax.ShapeDtypeStruct((BATCH, 256), jnp.float32),
        scratch_types=[
            pltpu.VMEM((RPW * N_SP,), jnp.int32),
            pltpu.VMEM((table.shape[0],), jnp.float32),
            pltpu.VMEM((CHUNK, 128), jnp.float32),
            pltpu.VMEM((CHUNK, 128), jnp.float32),
            pltpu.VMEM((CHUNK, 128), jnp.float32),
            pltpu.VMEM((CHUNK, 128), jnp.float32),
            pltpu.SemaphoreType.DMA,
            pltpu.SemaphoreType.DMA,
            pltpu.SemaphoreType.DMA,
            pltpu.SemaphoreType.DMA,
        ],
    )
    return k(sparse.reshape(-1), table, dp)[:, :D_OUT]


# speedup vs baseline: 2.3704x; 1.0574x over previous
import functools

import jax
import jax.numpy as jnp
from jax import lax
from jax.experimental import pallas as pl
from jax.experimental.pallas import tpu as pltpu
from jax.experimental.pallas import tpu_sc as plsc

EMB = 8
N_SP = 20
N_DN = 10
D_SP = N_SP * EMB
D_DN = N_DN * EMB
D_OUT = D_SP + D_DN
BATCH = 16384
NC = 2
NS = 16
NW = NC * NS
RPW = BATCH // NW
CHUNK = 128
GROUPS = CHUNK // 16
TB = 8192


def _dense_body(dense_ref, wt_ref, b_ref, out_ref):
    out_ref[...] = (
        jnp.dot(dense_ref[...], wt_ref[...],
                preferred_element_type=jnp.float32)
        + b_ref[...]
    )


D_CT1 = D_OUT - 128


def _sc_body(offs, sp_ref, tb_ref, dp_ref, out_ref, sp_v, tb_v,
             ob0a, ob0b, ob1a, ob1b, sem0a, sem0b, sem1a, sem1b,
             sem_in, sem_dp):
    cid = lax.axis_index("c")
    sid = lax.axis_index("s")
    wid = sid * NC + cid
    base = wid * RPW
    cp_sp = pltpu.async_copy(
        sp_ref.at[pl.ds(base * N_SP, RPW * N_SP)], sp_v, sem_in)
    cp_tb = pltpu.async_copy(tb_ref, tb_v, sem_in)
    iota = lax.iota(jnp.int32, 16)
    iota_sp = iota * N_SP

    def do_chunk(chunk, ct0, ct1, s0, s1, first):
        r0c = chunk * CHUNK
        rows = pl.ds(base + r0c, CHUNK)
        if not first:
            pltpu.make_async_copy(
                ct0, out_ref.at[rows, pl.ds(0, 128)], s0).wait()
            pltpu.make_async_copy(
                ct1, out_ref.at[rows, pl.ds(128, 128)], s1).wait()
        cp_dp = pltpu.async_copy(dp_ref.at[rows, :], ct1, sem_dp)
        if chunk == 0:
            cp_sp.wait()
            cp_tb.wait()

        def group(g, carry):
            r0 = r0c + g * 16
            rows_sp = iota_sp + r0 * N_SP
            rows_ob = iota + g * 16
            sidx8 = [plsc.load_gather(sp_v, [rows_sp + i]) * EMB
                     for i in range(N_SP)]
            for i in range(N_SP):
                t8 = sidx8[i]
                vs = [plsc.load_gather(tb_v, [t8 + (offs[i] * EMB + j)])
                      for j in range(EMB)]
                for j in range(EMB):
                    c = EMB * i + j
                    tgt = ct0 if c < 128 else ct1
                    ci = jnp.full((16,), c % 128, jnp.int32)
                    plsc.store_scatter(tgt, [rows_ob, ci], vs[j])
            return carry

        lax.fori_loop(0, GROUPS, group, 0)
        cp_dp.wait()
        pltpu.async_copy(ct0, out_ref.at[rows, pl.ds(0, 128)], s0)
        pltpu.async_copy(ct1, out_ref.at[rows, pl.ds(128, 128)], s1)

    bufs = [(ob0a, ob1a, sem0a, sem1a), (ob0b, ob1b, sem0b, sem1b)]
    n_chunks = RPW // CHUNK
    for c in range(n_chunks):
        ct0, ct1, s0, s1 = bufs[c % 2]
        do_chunk(c, ct0, ct1, s0, s1, c < 2)
    for c in range(n_chunks - 2, n_chunks):
        ct0, ct1, s0, s1 = bufs[c % 2]
        rows = pl.ds(base + c * CHUNK, CHUNK)
        pltpu.make_async_copy(ct0, out_ref.at[rows, pl.ds(0, 128)], s0).wait()
        pltpu.make_async_copy(
            ct1, out_ref.at[rows, pl.ds(128, 128)], s1).wait()


def kernel(sparse, dense, W, b, emb0, emb1, emb2, emb3, emb4, emb5, emb6,
           emb7, emb8, emb9, emb10, emb11, emb12, emb13, emb14, emb15,
           emb16, emb17, emb18, emb19):
    embs = [emb0, emb1, emb2, emb3, emb4, emb5, emb6, emb7, emb8, emb9,
            emb10, emb11, emb12, emb13, emb14, emb15, emb16, emb17, emb18,
            emb19]
    offs = []
    t = 0
    for e in embs:
        offs.append(t)
        t += e.shape[0]
    table = jnp.concatenate(embs, axis=0).reshape(-1)
    pad = (-table.shape[0]) % 128
    table = jnp.concatenate([table, jnp.zeros((pad,), jnp.float32)])

    lpad = D_SP - 128
    rpad = 128 - lpad - D_DN
    w_pad = jnp.concatenate([
        jnp.zeros((lpad, N_DN), jnp.float32), W,
        jnp.zeros((rpad, N_DN), jnp.float32)])
    b_pad = jnp.concatenate([
        jnp.zeros((lpad,), jnp.float32), b,
        jnp.zeros((rpad,), jnp.float32)])

    dp = pl.pallas_call(
        _dense_body,
        grid=(BATCH // TB,),
        in_specs=[
            pl.BlockSpec((TB, N_DN), lambda i: (i, 0)),
            pl.BlockSpec((N_DN, 128), lambda i: (0, 0)),
            pl.BlockSpec((1, 128), lambda i: (0, 0)),
        ],
        out_specs=pl.BlockSpec((TB, 128), lambda i: (i, 0)),
        out_shape=jax.ShapeDtypeStruct((BATCH, 128), jnp.float32),
    )(dense, w_pad.T, b_pad[None, :])

    mesh = plsc.VectorSubcoreMesh(core_axis_name="c", subcore_axis_name="s")
    k = pl.kernel(
        functools.partial(_sc_body, tuple(offs)),
        mesh=mesh,
        compiler_params=pltpu.CompilerParams(
            needs_layout_passes=False, use_tc_tiling_on_sc=True),
        out_type=jax.ShapeDtypeStruct((BATCH, 256), jnp.float32),
        scratch_types=[
            pltpu.VMEM((RPW * N_SP,), jnp.int32),
            pltpu.VMEM((table.shape[0],), jnp.float32),
            pltpu.VMEM((CHUNK, 128), jnp.float32),
            pltpu.VMEM((CHUNK, 128), jnp.float32),
            pltpu.VMEM((CHUNK, 128), jnp.float32),
            pltpu.VMEM((CHUNK, 128), jnp.float32),
            pltpu.SemaphoreType.DMA,
            pltpu.SemaphoreType.DMA,
            pltpu.SemaphoreType.DMA,
            pltpu.SemaphoreType.DMA,
            pltpu.SemaphoreType.DMA,
            pltpu.SemaphoreType.DMA,
        ],
    )
    return k(sparse.reshape(-1), table, dp)[:, :D_OUT]


# speedup vs baseline: 2.3707x; 1.0001x over previous
import functools

import jax
import jax.numpy as jnp
from jax import lax
from jax.experimental import pallas as pl
from jax.experimental.pallas import tpu as pltpu
from jax.experimental.pallas import tpu_sc as plsc

EMB = 8
N_SP = 20
N_DN = 10
D_SP = N_SP * EMB
D_DN = N_DN * EMB
D_OUT = D_SP + D_DN
BATCH = 16384
NC = 2
NS = 16
NW = NC * NS
RPW = BATCH // NW
CHUNK = 128
GROUPS = CHUNK // 16
TB = 8192


def _dense_body(dense_ref, wt_ref, b_ref, out_ref):
    out_ref[...] = (
        jnp.dot(dense_ref[...], wt_ref[...],
                preferred_element_type=jnp.float32)
        + b_ref[...]
    )


D_CT1 = D_OUT - 128


def _sc_body(offs, sp_ref, tb_ref, dp_ref, out_ref, sp_v, tb_v,
             ob0a, ob0b, ob1a, ob1b, sem0a, sem0b, sem1a, sem1b,
             sem_in, sem_dp):
    cid = lax.axis_index("c")
    sid = lax.axis_index("s")
    wid = sid * NC + cid
    base = wid * RPW
    cp_sp = pltpu.async_copy(
        sp_ref.at[pl.ds(base * N_SP, RPW * N_SP)], sp_v, sem_in)
    cp_tb = pltpu.async_copy(tb_ref, tb_v, sem_in)
    iota = lax.iota(jnp.int32, 16)
    iota_sp = iota * N_SP

    def do_chunk(chunk, ct0, ct1, s0, s1, first):
        r0c = chunk * CHUNK
        rows = pl.ds(base + r0c, CHUNK)
        if not first:
            pltpu.make_async_copy(
                ct0, out_ref.at[rows, pl.ds(0, 128)], s0).wait()
            pltpu.make_async_copy(
                ct1, out_ref.at[rows, pl.ds(128, 128)], s1).wait()
        cp_dp = pltpu.async_copy(dp_ref.at[rows, :], ct1, sem_dp)
        if chunk == 0:
            cp_sp.wait()
            cp_tb.wait()

        def make_group(lo, hi, tgt, coff):
            def group(g, carry):
                r0 = r0c + g * 16
                rows_sp = iota_sp + r0 * N_SP
                rows_ob = iota + g * 16
                sidx8 = [plsc.load_gather(sp_v, [rows_sp + i]) * EMB
                         for i in range(lo, hi)]
                for i in range(lo, hi):
                    t8 = sidx8[i - lo]
                    vs = [plsc.load_gather(tb_v, [t8 + (offs[i] * EMB + j)])
                          for j in range(EMB)]
                    for j in range(EMB):
                        ci = jnp.full((16,), EMB * i + j - coff, jnp.int32)
                        plsc.store_scatter(tgt, [rows_ob, ci], vs[j])
                return carry
            return group

        lax.fori_loop(0, GROUPS, make_group(0, 16, ct0, 0), 0)
        cp_dp.wait()
        lax.fori_loop(0, GROUPS, make_group(16, N_SP, ct1, 128), 0)
        pltpu.async_copy(ct0, out_ref.at[rows, pl.ds(0, 128)], s0)
        pltpu.async_copy(ct1, out_ref.at[rows, pl.ds(128, 128)], s1)

    bufs = [(ob0a, ob1a, sem0a, sem1a), (ob0b, ob1b, sem0b, sem1b)]
    n_chunks = RPW // CHUNK
    for c in range(n_chunks):
        ct0, ct1, s0, s1 = bufs[c % 2]
        do_chunk(c, ct0, ct1, s0, s1, c < 2)
    for c in range(n_chunks - 2, n_chunks):
        ct0, ct1, s0, s1 = bufs[c % 2]
        rows = pl.ds(base + c * CHUNK, CHUNK)
        pltpu.make_async_copy(ct0, out_ref.at[rows, pl.ds(0, 128)], s0).wait()
        pltpu.make_async_copy(
            ct1, out_ref.at[rows, pl.ds(128, 128)], s1).wait()


def kernel(sparse, dense, W, b, emb0, emb1, emb2, emb3, emb4, emb5, emb6,
           emb7, emb8, emb9, emb10, emb11, emb12, emb13, emb14, emb15,
           emb16, emb17, emb18, emb19):
    embs = [emb0, emb1, emb2, emb3, emb4, emb5, emb6, emb7, emb8, emb9,
            emb10, emb11, emb12, emb13, emb14, emb15, emb16, emb17, emb18,
            emb19]
    offs = []
    t = 0
    for e in embs:
        offs.append(t)
        t += e.shape[0]
    table = jnp.concatenate(embs, axis=0).reshape(-1)
    pad = (-table.shape[0]) % 128
    table = jnp.concatenate([table, jnp.zeros((pad,), jnp.float32)])

    lpad = D_SP - 128
    rpad = 128 - lpad - D_DN
    w_pad = jnp.concatenate([
        jnp.zeros((lpad, N_DN), jnp.float32), W,
        jnp.zeros((rpad, N_DN), jnp.float32)])
    b_pad = jnp.concatenate([
        jnp.zeros((lpad,), jnp.float32), b,
        jnp.zeros((rpad,), jnp.float32)])

    dp = pl.pallas_call(
        _dense_body,
        grid=(BATCH // TB,),
        in_specs=[
            pl.BlockSpec((TB, N_DN), lambda i: (i, 0)),
            pl.BlockSpec((N_DN, 128), lambda i: (0, 0)),
            pl.BlockSpec((1, 128), lambda i: (0, 0)),
        ],
        out_specs=pl.BlockSpec((TB, 128), lambda i: (i, 0)),
        out_shape=jax.ShapeDtypeStruct((BATCH, 128), jnp.float32),
    )(dense, w_pad.T, b_pad[None, :])

    mesh = plsc.VectorSubcoreMesh(core_axis_name="c", subcore_axis_name="s")
    k = pl.kernel(
        functools.partial(_sc_body, tuple(offs)),
        mesh=mesh,
        compiler_params=pltpu.CompilerParams(
            needs_layout_passes=False, use_tc_tiling_on_sc=True),
        out_type=jax.ShapeDtypeStruct((BATCH, 256), jnp.float32),
        scratch_types=[
            pltpu.VMEM((RPW * N_SP,), jnp.int32),
            pltpu.VMEM((table.shape[0],), jnp.float32),
            pltpu.VMEM((CHUNK, 128), jnp.float32),
            pltpu.VMEM((CHUNK, 128), jnp.float32),
            pltpu.VMEM((CHUNK, 128), jnp.float32),
            pltpu.VMEM((CHUNK, 128), jnp.float32),
            pltpu.SemaphoreType.DMA,
            pltpu.SemaphoreType.DMA,
            pltpu.SemaphoreType.DMA,
            pltpu.SemaphoreType.DMA,
            pltpu.SemaphoreType.DMA,
            pltpu.SemaphoreType.DMA,
        ],
    )
    return k(sparse.reshape(-1), table, dp)[:, :D_OUT]
